# baseline (speedup 1.0000x reference)
import functools

import jax
import jax.numpy as jnp
from jax.experimental import pallas as pl

N = 10000
E = 320000
FEAT = 128
N_RBF = 20
CUTOFF = 5.0


def _phi_body(h_ref, w1_ref, b1_ref, w2_ref, b2_ref, out_ref):
    h = h_ref[...]
    z = jnp.dot(h, w1_ref[...], preferred_element_type=jnp.float32) + b1_ref[...]
    z = z * jax.nn.sigmoid(z)
    out_ref[...] = (
        jnp.dot(z, w2_ref[...], preferred_element_type=jnp.float32) + b2_ref[...]
    )


def _compute_phi(h_i, W1, b1, W2, b2):
    BN = 1000
    grid = (N // BN,)
    return pl.pallas_call(
        _phi_body,
        grid=grid,
        in_specs=[
            pl.BlockSpec((BN, FEAT), lambda i: (i, 0)),
            pl.BlockSpec((FEAT, FEAT), lambda i: (0, 0)),
            pl.BlockSpec((1, FEAT), lambda i: (0, 0)),
            pl.BlockSpec((FEAT, 3 * FEAT), lambda i: (0, 0)),
            pl.BlockSpec((1, 3 * FEAT), lambda i: (0, 0)),
        ],
        out_specs=pl.BlockSpec((BN, 3 * FEAT), lambda i: (i, 0)),
        out_shape=jax.ShapeDtypeStruct((N, 3 * FEAT), jnp.float32),
    )(h_i, W1, b1.reshape(1, FEAT), W2, b2.reshape(1, -1))


def _dist_body(d_ref, n_ref, wd_ref, bd_ref, out_ref):
    d = d_ref[...]
    n = n_ref[...]
    rbf = jnp.sin(n * jnp.pi * d / CUTOFF) / d
    e_feats = jnp.dot(rbf, wd_ref[...], preferred_element_type=jnp.float32) + bd_ref[...]
    env = 0.5 * (jnp.cos(jnp.pi * d / CUTOFF) + 1.0) * (d < CUTOFF).astype(jnp.float32)
    out_ref[...] = e_feats * env


def _compute_dist_emb(d_ij, Wd, bd):
    BE = 2000
    grid = (E // BE,)
    return pl.pallas_call(
        _dist_body,
        grid=grid,
        in_specs=[
            pl.BlockSpec((BE, 1), lambda i: (i, 0)),
            pl.BlockSpec((1, N_RBF), lambda i: (0, 0)),
            pl.BlockSpec((N_RBF, 3 * FEAT), lambda i: (0, 0)),
            pl.BlockSpec((1, 3 * FEAT), lambda i: (0, 0)),
        ],
        out_specs=pl.BlockSpec((BE, 3 * FEAT), lambda i: (i, 0)),
        out_shape=jax.ShapeDtypeStruct((E, 3 * FEAT), jnp.float32),
    )(d_ij.reshape(E, 1),
      jnp.arange(1, N_RBF + 1, dtype=jnp.float32).reshape(1, N_RBF), Wd,
      bd.reshape(1, -1))


def kernel(h_i, v_i, d_ij, unit_r_ij, nbrs, W1, b1, W2, b2, Wd, bd):
    phi = _compute_phi(h_i, W1, b1, W2, b2)
    dist_emb = _compute_dist_emb(d_ij, Wd, bd)

    src = nbrs[:, 1]
    dst = nbrs[:, 0]
    edge_inv = jnp.take(phi, src, axis=0) * dist_emb
    edge_inv = edge_inv.reshape(E, 3, FEAT)
    filter1 = edge_inv[:, 0]
    filter2 = edge_inv[:, 1]
    filter3 = edge_inv[:, 2]
    v_gather = jnp.take(v_i, src, axis=0)
    dv = filter1[:, :, None] * unit_r_ij[:, None, :] + filter2[:, :, None] * v_gather
    dh_i = jax.ops.segment_sum(filter3, dst, num_segments=N)
    dv_i = jax.ops.segment_sum(dv, dst, num_segments=N)
    return (dh_i, dv_i)
